# baseline (speedup 1.0000x reference)
import jax
import jax.numpy as jnp
from jax.experimental import pallas as pl
from jax.experimental.pallas import tpu as pltpu


def _sda_body(h_ref, ctx_ref, w_in_ref, w_out_ref, out_ref, attn_ref):
    h = h_ref[...]
    ctx = ctx_ref[...]

    target = jnp.dot(h, w_in_ref[...],
                     preferred_element_type=jnp.float32)

    logits = jnp.sum(ctx * target[:, None, :], axis=-1)

    m = jnp.max(logits, axis=-1, keepdims=True)
    e = jnp.exp(logits - m)
    attn = e * (1.0 / jnp.sum(e, axis=-1, keepdims=True))

    wctx = jnp.sum(attn[:, :, None] * ctx, axis=1)

    cat = jnp.concatenate([wctx, h], axis=-1)
    pre = jnp.dot(cat, w_out_ref[...],
                  preferred_element_type=jnp.float32)
    out_ref[...] = jnp.tanh(pre).astype(out_ref.dtype)
    attn_ref[...] = attn.astype(attn_ref.dtype)


def _pick_tile(B):
    for cand in (240, 192, 160, 128, 120, 96, 64, 32, 16, 8):
        if B % cand == 0 and (B // cand) % 2 == 0:
            return cand
    return B


def kernel(h, context, w_in, w_out_c, w_out_h):
    B, Dq = h.shape
    _, S, Dc = context.shape
    tile_b = _pick_tile(B)

    w_out = jnp.concatenate([w_out_c, w_out_h], axis=0)

    h_tilde, attn = pl.pallas_call(
        _sda_body,
        out_shape=(jax.ShapeDtypeStruct((B, Dq), h.dtype),
                   jax.ShapeDtypeStruct((B, S), jnp.float32)),
        grid=(B // tile_b,),
        in_specs=[
            pl.BlockSpec((tile_b, Dq), lambda i: (i, 0)),
            pl.BlockSpec((tile_b, S, Dc), lambda i: (i, 0, 0)),
            pl.BlockSpec((Dq, Dc), lambda i: (0, 0)),
            pl.BlockSpec((Dc + Dq, Dq), lambda i: (0, 0)),
        ],
        out_specs=(pl.BlockSpec((tile_b, Dq), lambda i: (i, 0)),
                   pl.BlockSpec((tile_b, S), lambda i: (i, 0))),
        compiler_params=pltpu.CompilerParams(
            dimension_semantics=("parallel",),
        ),
    )(h, context, w_in, w_out)
    return h_tilde, attn


# speedup vs baseline: 1.3916x; 1.3916x over previous
import jax
import jax.numpy as jnp
from jax.experimental import pallas as pl
from jax.experimental.pallas import tpu as pltpu


def _sda_body(h_ref, ctx_ref, w_in_ref, w_out_ref, out_ref, attn_ref,
              wctx_ref, cb_ref):
    tb, S, Dc = ctx_ref.shape
    h = h_ref[...]

    target = jnp.dot(h, w_in_ref[...],
                     preferred_element_type=jnp.float32)

    eye8 = jnp.eye(8, dtype=jnp.float32)
    nch = tb // 8

    cb_ref[...] = ctx_ref[...].reshape(tb * S, Dc).astype(jnp.bfloat16)

    for g in range(nch):
        c2 = ctx_ref[g * 8:(g + 1) * 8, :, :].reshape(8 * S, Dc)
        tg = target[g * 8:(g + 1) * 8, :]
        r = jax.lax.dot_general(
            tg, c2,
            (((1,), (1,)), ((), ())),
            preferred_element_type=jnp.float32)
        r3 = r.reshape(8, 8, S)
        attn_ref[g * 8:(g + 1) * 8, :] = jnp.sum(r3 * eye8[:, :, None], axis=0)

    logits = attn_ref[...]
    m = jnp.max(logits, axis=-1, keepdims=True)
    e = jnp.exp(logits - m)
    attn = e * (1.0 / jnp.sum(e, axis=-1, keepdims=True))
    attn_ref[...] = attn

    for g in range(nch):
        c2 = cb_ref[g * 8 * S:(g + 1) * 8 * S, :]
        ag = attn[g * 8:(g + 1) * 8, :]
        a_bd = (ag[:, None, :] * eye8[:, :, None]).reshape(8, 8 * S)
        wctx_ref[g * 8:(g + 1) * 8, :] = jnp.dot(
            a_bd.astype(jnp.bfloat16), c2,
            preferred_element_type=jnp.float32)

    cat = jnp.concatenate([wctx_ref[...], h], axis=-1)
    pre = jnp.dot(cat, w_out_ref[...],
                  preferred_element_type=jnp.float32)
    out_ref[...] = jnp.tanh(pre).astype(out_ref.dtype)


def _pick_tile(B):
    for cand in (240, 192, 160, 128, 120, 96, 64, 32, 16, 8):
        if B % cand == 0 and (B // cand) % 2 == 0:
            return cand
    return B


def kernel(h, context, w_in, w_out_c, w_out_h):
    B, Dq = h.shape
    _, S, Dc = context.shape
    tile_b = _pick_tile(B)

    w_out = jnp.concatenate([w_out_c, w_out_h], axis=0)

    h_tilde, attn = pl.pallas_call(
        _sda_body,
        out_shape=(jax.ShapeDtypeStruct((B, Dq), h.dtype),
                   jax.ShapeDtypeStruct((B, S), jnp.float32)),
        grid=(B // tile_b,),
        in_specs=[
            pl.BlockSpec((tile_b, Dq), lambda i: (i, 0)),
            pl.BlockSpec((tile_b, S, Dc), lambda i: (i, 0, 0)),
            pl.BlockSpec((Dq, Dc), lambda i: (0, 0)),
            pl.BlockSpec((Dc + Dq, Dq), lambda i: (0, 0)),
        ],
        out_specs=(pl.BlockSpec((tile_b, Dq), lambda i: (i, 0)),
                   pl.BlockSpec((tile_b, S), lambda i: (i, 0))),
        scratch_shapes=[pltpu.VMEM((tile_b, Dc), jnp.float32),
                        pltpu.VMEM((tile_b * S, Dc), jnp.bfloat16)],
        compiler_params=pltpu.CompilerParams(
            dimension_semantics=("parallel",),
        ),
    )(h, context, w_in, w_out)
    return h_tilde, attn
